# speedup vs baseline: 975.4815x; 975.4815x over previous
import jax
import jax.numpy as jnp
from jax.experimental import pallas as pl

_SDR = 2048
_CONT = 384
_ROWS_OUT = 4672
_LAST_DATA_ROW = 4096
_BLK = 64
_NBLK = _ROWS_OUT // _BLK


def _src_block(b):
    return jnp.where(
        b == 0, 63,
        jnp.where(b <= 7, b + 55,
                  jnp.where(b == 8, 55,
                            jnp.where(b <= 63, b - 9, 55))))


def _body(sdr_ref, cont_ref, out_ref):
    b = pl.program_id(0)
    rows = jax.lax.broadcasted_iota(jnp.int32, (_BLK, 1), 0) + b * _BLK
    valid = rows <= _LAST_DATA_ROW
    out_ref[:, :_SDR] = jnp.where(valid, sdr_ref[...], 0.0)
    out_ref[:, _SDR:] = jnp.where(valid, cont_ref[...], 0.0)


def kernel(sdrs, contents):
    return pl.pallas_call(
        _body,
        grid=(_NBLK,),
        in_specs=[
            pl.BlockSpec((_BLK, _SDR), lambda b: (_src_block(b), 0)),
            pl.BlockSpec((_BLK, _CONT), lambda b: (_src_block(b), 0)),
        ],
        out_specs=pl.BlockSpec((_BLK, _SDR + _CONT), lambda b: (b, 0)),
        out_shape=jax.ShapeDtypeStruct((_ROWS_OUT, _SDR + _CONT), jnp.float32),
    )(sdrs, contents)
